# baseline (speedup 1.0000x reference)
import functools

import jax
import jax.numpy as jnp
from jax import lax
from jax.experimental import pallas as pl
from jax.experimental.pallas import tpu as pltpu
from jax.experimental.pallas import tpu_sc as plsc

ROWS = 100001
DESC_DIM = 128
TSE_DIM = 64
BATCH = 4096
HIST = 50
TOTAL = BATCH * HIST

ROW_BLOCK = 2048

NUM_CORES = 2
NUM_SUBCORES = 16
NW = NUM_CORES * NUM_SUBCORES
CHUNK = BATCH // NW

L_STRIDE = TSE_DIM * BATCH
I_STRIDE = 8 * BATCH
TILE_W = 8 * CHUNK


def _fuse_body(sem_ref, wt_ref, b_ref, idt_ref, out_ref):
    out_ref[...] = (
        jnp.dot(sem_ref[...], wt_ref[...], preferred_element_type=jnp.float32)
        + b_ref[...]
        + idt_ref[...].T
    )


def _fused_table(sem, wt, b2, idt):
    grid = (pl.cdiv(ROWS, ROW_BLOCK),)
    return pl.pallas_call(
        _fuse_body,
        grid=grid,
        in_specs=[
            pl.BlockSpec((ROW_BLOCK, DESC_DIM), lambda i: (i, 0)),
            pl.BlockSpec((DESC_DIM, TSE_DIM), lambda i: (0, 0)),
            pl.BlockSpec((1, TSE_DIM), lambda i: (0, 0)),
            pl.BlockSpec((TSE_DIM, ROW_BLOCK), lambda i: (0, i)),
        ],
        out_specs=pl.BlockSpec((ROW_BLOCK, TSE_DIM), lambda i: (i, 0)),
        out_shape=jax.ShapeDtypeStruct((ROWS, TSE_DIM), jnp.float32),
    )(sem, wt, b2, idt)


def _gather(idx3, table):
    mesh = plsc.VectorSubcoreMesh(core_axis_name="c", subcore_axis_name="s")

    @functools.partial(
        pl.kernel,
        mesh=mesh,
        compiler_params=pltpu.CompilerParams(
            use_tc_tiling_on_sc=False, needs_layout_passes=False
        ),
        out_type=jax.ShapeDtypeStruct((TOTAL * TSE_DIM,), jnp.float32),
        scratch_types=[
            pltpu.VMEM((HIST, CHUNK), jnp.int32),
            pltpu.VMEM((CHUNK, TSE_DIM), jnp.float32),
            pltpu.VMEM((CHUNK, TSE_DIM), jnp.float32),
            pltpu.VMEM((8 * TILE_W,), jnp.float32),
            pltpu.VMEM((8 * TILE_W,), jnp.float32),
            pltpu.SemaphoreType.DMA,
            pltpu.SemaphoreType.DMA,
            pltpu.SemaphoreType.DMA,
            pltpu.SemaphoreType.DMA,
        ],
    )
    def k(idx_hbm, table_hbm, out_hbm, idx_v, g0, g1, y0, y1,
          gsem0, gsem1, wsem0, wsem1):
        wid = lax.axis_index("s") * NUM_CORES + lax.axis_index("c")
        pltpu.sync_copy(idx_hbm.at[wid], idx_v)

        iotas = [lax.iota(jnp.int32, 16) + seg * 16 for seg in range(8)]

        def transpose(g, y):
            def ibody(i, _):
                for r in range(8):
                    o = i * 8 + r
                    cols = jnp.zeros((16,), jnp.int32) + o
                    for seg in range(8):
                        v = plsc.load_gather(g, [iotas[seg], cols])
                        y[pl.ds(i * TILE_W + r * CHUNK + seg * 16, 16)] = v
                return 0

            lax.fori_loop(0, 8, ibody, 0)

        def issue_writes(y, l, wsem):
            base = l * L_STRIDE + wid * TILE_W
            for i in range(8):
                pltpu.async_copy(
                    y.at[pl.ds(i * TILE_W, TILE_W)],
                    out_hbm.at[pl.ds(base + i * I_STRIDE, TILE_W)],
                    wsem,
                )

        def drain_writes(y, wsem):
            for _ in range(8):
                pltpu.make_async_copy(
                    y.at[pl.ds(0, TILE_W)],
                    out_hbm.at[pl.ds(0, TILE_W)],
                    wsem,
                ).wait()

        pltpu.async_copy(table_hbm.at[idx_v.at[0]], g0, gsem0)

        def pair_body(p, _):
            l0 = p * 2
            pltpu.make_async_copy(table_hbm.at[idx_v.at[l0]], g0, gsem0).wait()
            pltpu.async_copy(table_hbm.at[idx_v.at[l0 + 1]], g1, gsem1)

            @pl.when(p >= 1)
            def _():
                drain_writes(y0, wsem0)

            transpose(g0, y0)
            issue_writes(y0, l0, wsem0)

            pltpu.make_async_copy(
                table_hbm.at[idx_v.at[l0 + 1]], g1, gsem1
            ).wait()

            @pl.when(l0 + 2 < HIST)
            def _():
                pltpu.async_copy(table_hbm.at[idx_v.at[l0 + 2]], g0, gsem0)

            @pl.when(p >= 1)
            def _():
                drain_writes(y1, wsem1)

            transpose(g1, y1)
            issue_writes(y1, l0 + 1, wsem1)

            return 0

        lax.fori_loop(0, HIST // 2, pair_body, 0)
        drain_writes(y0, wsem0)
        drain_writes(y1, wsem1)

    return k(idx3, table)


def kernel(product_ids, semantic_table, fcn_W, fcn_b, id_table):
    wt = fcn_W.T
    b2 = fcn_b.reshape(1, TSE_DIM)
    idt = id_table.T
    table = _fused_table(semantic_table, wt, b2, idt)
    idx3 = (
        product_ids.astype(jnp.int32).T.reshape(HIST, NW, CHUNK)
        .transpose(1, 0, 2)
    )
    out = _gather(idx3, table)
    return (
        out.reshape(HIST, 8, NW, 8, CHUNK)
        .transpose(2, 4, 0, 1, 3)
        .reshape(BATCH, HIST, TSE_DIM)
    )


# speedup vs baseline: 1.1633x; 1.1633x over previous
import functools

import jax
import jax.numpy as jnp
from jax import lax
from jax.experimental import pallas as pl
from jax.experimental.pallas import tpu as pltpu
from jax.experimental.pallas import tpu_sc as plsc

ROWS = 100001
DESC_DIM = 128
TSE_DIM = 64
BATCH = 4096
HIST = 50
TOTAL = BATCH * HIST

ROW_BLOCK = 2048

NUM_CORES = 2
NUM_SUBCORES = 16
NW = NUM_CORES * NUM_SUBCORES
CHUNK = BATCH // NW

L_STRIDE = TSE_DIM * BATCH
I_STRIDE = 8 * BATCH
TILE_W = 8 * CHUNK


def _fuse_body(sem_ref, wt_ref, b_ref, idt_ref, out_ref):
    out_ref[...] = (
        jnp.dot(sem_ref[...], wt_ref[...], preferred_element_type=jnp.float32)
        + b_ref[...]
        + idt_ref[...].T
    )


def _fused_table(sem, wt, b2, idt):
    grid = (pl.cdiv(ROWS, ROW_BLOCK),)
    return pl.pallas_call(
        _fuse_body,
        grid=grid,
        in_specs=[
            pl.BlockSpec((ROW_BLOCK, DESC_DIM), lambda i: (i, 0)),
            pl.BlockSpec((DESC_DIM, TSE_DIM), lambda i: (0, 0)),
            pl.BlockSpec((1, TSE_DIM), lambda i: (0, 0)),
            pl.BlockSpec((TSE_DIM, ROW_BLOCK), lambda i: (0, i)),
        ],
        out_specs=pl.BlockSpec((ROW_BLOCK, TSE_DIM), lambda i: (i, 0)),
        out_shape=jax.ShapeDtypeStruct((ROWS, TSE_DIM), jnp.float32),
    )(sem, wt, b2, idt)


def _gather(idx3, table):
    mesh = plsc.VectorSubcoreMesh(core_axis_name="c", subcore_axis_name="s")

    @functools.partial(
        pl.kernel,
        mesh=mesh,
        compiler_params=pltpu.CompilerParams(
            use_tc_tiling_on_sc=False, needs_layout_passes=False
        ),
        out_type=jax.ShapeDtypeStruct((TOTAL * TSE_DIM,), jnp.float32),
        scratch_types=[
            pltpu.VMEM((HIST, CHUNK), jnp.int32),
            pltpu.VMEM((CHUNK, TSE_DIM), jnp.float32),
            pltpu.VMEM((CHUNK, TSE_DIM), jnp.float32),
            pltpu.VMEM((8 * TILE_W,), jnp.float32),
            pltpu.VMEM((8 * TILE_W,), jnp.float32),
            pltpu.SemaphoreType.DMA,
            pltpu.SemaphoreType.DMA,
            pltpu.SemaphoreType.DMA,
            pltpu.SemaphoreType.DMA,
        ],
    )
    def k(idx_hbm, table_hbm, out_hbm, idx_v, g0, g1, y0, y1,
          gsem0, gsem1, wsem0, wsem1):
        wid = lax.axis_index("s") * NUM_CORES + lax.axis_index("c")
        pltpu.sync_copy(idx_hbm.at[wid], idx_v)

        iota = lax.iota(jnp.int32, 16)
        base_pat = (iota // 8) * TILE_W + (iota % 8) * CHUNK
        pats = [base_pat + seg * 2 * TILE_W for seg in range(4)]

        def transpose(g, y):
            def cbody(p, _):
                c0 = p * 8
                for dc in range(8):
                    c = c0 + dc
                    for seg in range(4):
                        v = g[c, pl.ds(seg * 16, 16)]
                        plsc.store_scatter(y, [pats[seg] + c], v)
                return 0

            lax.fori_loop(0, CHUNK // 8, cbody, 0)

        def issue_writes(y, l, wsem):
            base = l * L_STRIDE + wid * TILE_W
            for i in range(8):
                pltpu.async_copy(
                    y.at[pl.ds(i * TILE_W, TILE_W)],
                    out_hbm.at[pl.ds(base + i * I_STRIDE, TILE_W)],
                    wsem,
                )

        def drain_writes(y, wsem):
            for _ in range(8):
                pltpu.make_async_copy(
                    y.at[pl.ds(0, TILE_W)],
                    out_hbm.at[pl.ds(0, TILE_W)],
                    wsem,
                ).wait()

        pltpu.async_copy(table_hbm.at[idx_v.at[0]], g0, gsem0)

        def pair_body(p, _):
            l0 = p * 2
            pltpu.make_async_copy(table_hbm.at[idx_v.at[l0]], g0, gsem0).wait()
            pltpu.async_copy(table_hbm.at[idx_v.at[l0 + 1]], g1, gsem1)

            @pl.when(p >= 1)
            def _():
                drain_writes(y0, wsem0)

            transpose(g0, y0)
            issue_writes(y0, l0, wsem0)

            pltpu.make_async_copy(
                table_hbm.at[idx_v.at[l0 + 1]], g1, gsem1
            ).wait()

            @pl.when(l0 + 2 < HIST)
            def _():
                pltpu.async_copy(table_hbm.at[idx_v.at[l0 + 2]], g0, gsem0)

            @pl.when(p >= 1)
            def _():
                drain_writes(y1, wsem1)

            transpose(g1, y1)
            issue_writes(y1, l0 + 1, wsem1)

            return 0

        lax.fori_loop(0, HIST // 2, pair_body, 0)
        drain_writes(y0, wsem0)
        drain_writes(y1, wsem1)

    return k(idx3, table)


def kernel(product_ids, semantic_table, fcn_W, fcn_b, id_table):
    wt = fcn_W.T
    b2 = fcn_b.reshape(1, TSE_DIM)
    idt = id_table.T
    table = _fused_table(semantic_table, wt, b2, idt)
    idx3 = (
        product_ids.astype(jnp.int32).T.reshape(HIST, NW, CHUNK)
        .transpose(1, 0, 2)
    )
    out = _gather(idx3, table)
    return (
        out.reshape(HIST, 8, NW, 8, CHUNK)
        .transpose(2, 4, 0, 1, 3)
        .reshape(BATCH, HIST, TSE_DIM)
    )


# speedup vs baseline: 1.1633x; 1.0001x over previous
import functools

import jax
import jax.numpy as jnp
from jax import lax
from jax.experimental import pallas as pl
from jax.experimental.pallas import tpu as pltpu
from jax.experimental.pallas import tpu_sc as plsc

ROWS = 100001
DESC_DIM = 128
TSE_DIM = 64
BATCH = 4096
HIST = 50
TOTAL = BATCH * HIST

ROW_BLOCK = 2048

NUM_CORES = 2
NUM_SUBCORES = 16
NW = NUM_CORES * NUM_SUBCORES
CHUNK = BATCH // NW

L_STRIDE = TSE_DIM * BATCH
I_STRIDE = 8 * BATCH
TILE_W = 8 * CHUNK


def _fuse_body(sem_ref, wt_ref, b_ref, idt_ref, out_ref):
    out_ref[...] = (
        jnp.dot(sem_ref[...], wt_ref[...], preferred_element_type=jnp.float32)
        + b_ref[...]
        + idt_ref[...].T
    )


def _fused_table(sem, wt, b2, idt):
    grid = (pl.cdiv(ROWS, ROW_BLOCK),)
    return pl.pallas_call(
        _fuse_body,
        grid=grid,
        in_specs=[
            pl.BlockSpec((ROW_BLOCK, DESC_DIM), lambda i: (i, 0)),
            pl.BlockSpec((DESC_DIM, TSE_DIM), lambda i: (0, 0)),
            pl.BlockSpec((1, TSE_DIM), lambda i: (0, 0)),
            pl.BlockSpec((TSE_DIM, ROW_BLOCK), lambda i: (0, i)),
        ],
        out_specs=pl.BlockSpec((ROW_BLOCK, TSE_DIM), lambda i: (i, 0)),
        out_shape=jax.ShapeDtypeStruct((ROWS, TSE_DIM), jnp.float32),
    )(sem, wt, b2, idt)


def _gather(idx3, table):
    mesh = plsc.VectorSubcoreMesh(core_axis_name="c", subcore_axis_name="s")

    @functools.partial(
        pl.kernel,
        mesh=mesh,
        compiler_params=pltpu.CompilerParams(
            use_tc_tiling_on_sc=False, needs_layout_passes=False, disable_bounds_checks=True
        ),
        out_type=jax.ShapeDtypeStruct((TOTAL * TSE_DIM,), jnp.float32),
        scratch_types=[
            pltpu.VMEM((HIST, CHUNK), jnp.int32),
            pltpu.VMEM((CHUNK, TSE_DIM), jnp.float32),
            pltpu.VMEM((CHUNK, TSE_DIM), jnp.float32),
            pltpu.VMEM((8 * TILE_W,), jnp.float32),
            pltpu.VMEM((8 * TILE_W,), jnp.float32),
            pltpu.SemaphoreType.DMA,
            pltpu.SemaphoreType.DMA,
            pltpu.SemaphoreType.DMA,
            pltpu.SemaphoreType.DMA,
        ],
    )
    def k(idx_hbm, table_hbm, out_hbm, idx_v, g0, g1, y0, y1,
          gsem0, gsem1, wsem0, wsem1):
        wid = lax.axis_index("s") * NUM_CORES + lax.axis_index("c")
        pltpu.sync_copy(idx_hbm.at[wid], idx_v)

        iota = lax.iota(jnp.int32, 16)
        base_pat = (iota // 8) * TILE_W + (iota % 8) * CHUNK
        pats = [base_pat + seg * 2 * TILE_W for seg in range(4)]

        def transpose(g, y):
            def cbody(p, _):
                c0 = p * 16
                for dc in range(16):
                    c = c0 + dc
                    for seg in range(4):
                        v = g[c, pl.ds(seg * 16, 16)]
                        plsc.store_scatter(y, [pats[seg] + c], v)
                return 0

            lax.fori_loop(0, CHUNK // 16, cbody, 0)

        def issue_writes(y, l, wsem):
            base = l * L_STRIDE + wid * TILE_W
            for i in range(8):
                pltpu.async_copy(
                    y.at[pl.ds(i * TILE_W, TILE_W)],
                    out_hbm.at[pl.ds(base + i * I_STRIDE, TILE_W)],
                    wsem,
                )

        def drain_writes(y, wsem):
            for _ in range(8):
                pltpu.make_async_copy(
                    y.at[pl.ds(0, TILE_W)],
                    out_hbm.at[pl.ds(0, TILE_W)],
                    wsem,
                ).wait()

        pltpu.async_copy(table_hbm.at[idx_v.at[0]], g0, gsem0)

        def pair_body(p, _):
            l0 = p * 2
            pltpu.make_async_copy(table_hbm.at[idx_v.at[l0]], g0, gsem0).wait()
            pltpu.async_copy(table_hbm.at[idx_v.at[l0 + 1]], g1, gsem1)

            @pl.when(p >= 1)
            def _():
                drain_writes(y0, wsem0)

            transpose(g0, y0)
            issue_writes(y0, l0, wsem0)

            pltpu.make_async_copy(
                table_hbm.at[idx_v.at[l0 + 1]], g1, gsem1
            ).wait()

            @pl.when(l0 + 2 < HIST)
            def _():
                pltpu.async_copy(table_hbm.at[idx_v.at[l0 + 2]], g0, gsem0)

            @pl.when(p >= 1)
            def _():
                drain_writes(y1, wsem1)

            transpose(g1, y1)
            issue_writes(y1, l0 + 1, wsem1)

            return 0

        lax.fori_loop(0, HIST // 2, pair_body, 0)
        drain_writes(y0, wsem0)
        drain_writes(y1, wsem1)

    return k(idx3, table)


def kernel(product_ids, semantic_table, fcn_W, fcn_b, id_table):
    wt = fcn_W.T
    b2 = fcn_b.reshape(1, TSE_DIM)
    idt = id_table.T
    table = _fused_table(semantic_table, wt, b2, idt)
    idx3 = (
        product_ids.astype(jnp.int32).T.reshape(HIST, NW, CHUNK)
        .transpose(1, 0, 2)
    )
    out = _gather(idx3, table)
    return (
        out.reshape(HIST, 8, NW, 8, CHUNK)
        .transpose(2, 4, 0, 1, 3)
        .reshape(BATCH, HIST, TSE_DIM)
    )


# speedup vs baseline: 1.5878x; 1.3648x over previous
import functools

import jax
import jax.numpy as jnp
from jax import lax
from jax.experimental import pallas as pl
from jax.experimental.pallas import tpu as pltpu
from jax.experimental.pallas import tpu_sc as plsc

ROWS = 100001
DESC_DIM = 128
TSE_DIM = 64
BATCH = 4096
HIST = 50
TOTAL = BATCH * HIST

ROW_BLOCK = 2048

NUM_CORES = 2
NUM_SUBCORES = 16
NW = NUM_CORES * NUM_SUBCORES
PER_W = TOTAL // NW
CHUNK = 128
NCHUNK = PER_W // CHUNK


def _fuse_body(sem_ref, wt_ref, b_ref, idt_ref, out_ref):
    out_ref[:, pl.ds(0, TSE_DIM)] = (
        jnp.dot(sem_ref[...], wt_ref[...], preferred_element_type=jnp.float32)
        + b_ref[...]
        + idt_ref[...].T
    )


def _fused_table(sem, wt, b2, idt):
    grid = (pl.cdiv(ROWS, ROW_BLOCK),)
    return pl.pallas_call(
        _fuse_body,
        grid=grid,
        in_specs=[
            pl.BlockSpec((ROW_BLOCK, DESC_DIM), lambda i: (i, 0)),
            pl.BlockSpec((DESC_DIM, TSE_DIM), lambda i: (0, 0)),
            pl.BlockSpec((1, TSE_DIM), lambda i: (0, 0)),
            pl.BlockSpec((TSE_DIM, ROW_BLOCK), lambda i: (0, i)),
        ],
        out_specs=pl.BlockSpec((ROW_BLOCK, 2 * TSE_DIM), lambda i: (i, 0)),
        out_shape=jax.ShapeDtypeStruct((ROWS, 2 * TSE_DIM), jnp.float32),
    )(sem, wt, b2, idt)


def _gather(idx3, table):
    mesh = plsc.VectorSubcoreMesh(core_axis_name="c", subcore_axis_name="s")

    @functools.partial(
        pl.kernel,
        mesh=mesh,
        compiler_params=pltpu.CompilerParams(use_tc_tiling_on_sc=False),
        out_type=jax.ShapeDtypeStruct((TOTAL, TSE_DIM), jnp.float32),
        scratch_types=[
            pltpu.VMEM((NCHUNK, CHUNK), jnp.int32),
            pltpu.VMEM((CHUNK, TSE_DIM), jnp.float32),
            pltpu.VMEM((CHUNK, TSE_DIM), jnp.float32),
            pltpu.SemaphoreType.DMA,
            pltpu.SemaphoreType.DMA,
        ],
    )
    def k(idx_hbm, table_hbm, out_hbm, idx_v, buf0, buf1, gsem0, gsem1):
        wid = lax.axis_index("s") * NUM_CORES + lax.axis_index("c")
        base = wid * PER_W
        pltpu.sync_copy(idx_hbm.at[wid], idx_v)

        pltpu.async_copy(table_hbm.at[idx_v.at[0]], buf0, gsem0)

        def pair_body(p, _):
            j0 = p * 2
            pltpu.make_async_copy(table_hbm.at[idx_v.at[j0]], buf0, gsem0).wait()

            @pl.when(j0 + 1 < NCHUNK)
            def _():
                pltpu.async_copy(table_hbm.at[idx_v.at[j0 + 1]], buf1, gsem1)

            pltpu.sync_copy(buf0, out_hbm.at[pl.ds(base + j0 * CHUNK, CHUNK)])

            @pl.when(j0 + 1 < NCHUNK)
            def _():
                pltpu.make_async_copy(
                    table_hbm.at[idx_v.at[j0 + 1]], buf1, gsem1
                ).wait()

                @pl.when(j0 + 2 < NCHUNK)
                def _():
                    pltpu.async_copy(table_hbm.at[idx_v.at[j0 + 2]], buf0, gsem0)

                pltpu.sync_copy(
                    buf1, out_hbm.at[pl.ds(base + (j0 + 1) * CHUNK, CHUNK)]
                )

            return 0

        lax.fori_loop(0, (NCHUNK + 1) // 2, pair_body, 0)

    return k(idx3, table)


def kernel(product_ids, semantic_table, fcn_W, fcn_b, id_table):
    wt = fcn_W.T
    b2 = fcn_b.reshape(1, TSE_DIM)
    idt = id_table.T
    table = _fused_table(semantic_table, wt, b2, idt)
    table2 = table.reshape(2 * ROWS, TSE_DIM)
    idx3 = (product_ids.astype(jnp.int32) * 2).reshape(NW, NCHUNK, CHUNK)
    out = _gather(idx3, table2)
    return out.reshape(BATCH, HIST, TSE_DIM)


# speedup vs baseline: 2.4251x; 1.5273x over previous
import functools

import jax
import jax.numpy as jnp
from jax import lax
from jax.experimental import pallas as pl
from jax.experimental.pallas import tpu as pltpu
from jax.experimental.pallas import tpu_sc as plsc

ROWS = 100001
DESC_DIM = 128
TSE_DIM = 64
BATCH = 4096
HIST = 50
TOTAL = BATCH * HIST

ROW_BLOCK = 2048

NUM_CORES = 2
NUM_SUBCORES = 16
NW = NUM_CORES * NUM_SUBCORES
CHUNK = BATCH // NW

L_STRIDE = TSE_DIM * BATCH
I_STRIDE = 8 * BATCH
TILE_W = 8 * CHUNK
Y_STRIDE = CHUNK + 7


def _fuse_body(sem_ref, wt_ref, b_ref, idt_ref, out_ref):
    out_ref[:, pl.ds(0, TSE_DIM)] = (
        jnp.dot(sem_ref[...], wt_ref[...], preferred_element_type=jnp.float32)
        + b_ref[...]
        + idt_ref[...].T
    )


def _fused_table(sem, wt, b2, idt):
    grid = (pl.cdiv(ROWS, ROW_BLOCK),)
    return pl.pallas_call(
        _fuse_body,
        grid=grid,
        in_specs=[
            pl.BlockSpec((ROW_BLOCK, DESC_DIM), lambda i: (i, 0)),
            pl.BlockSpec((DESC_DIM, TSE_DIM), lambda i: (0, 0)),
            pl.BlockSpec((1, TSE_DIM), lambda i: (0, 0)),
            pl.BlockSpec((TSE_DIM, ROW_BLOCK), lambda i: (0, i)),
        ],
        out_specs=pl.BlockSpec((ROW_BLOCK, 2 * TSE_DIM), lambda i: (i, 0)),
        out_shape=jax.ShapeDtypeStruct((ROWS, 2 * TSE_DIM), jnp.float32),
    )(sem, wt, b2, idt)


def _gather(idx3, table):
    mesh = plsc.VectorSubcoreMesh(core_axis_name="c", subcore_axis_name="s")

    @functools.partial(
        pl.kernel,
        mesh=mesh,
        compiler_params=pltpu.CompilerParams(
            use_tc_tiling_on_sc=False,
            needs_layout_passes=False,
            disable_bounds_checks=True,
        ),
        out_type=jax.ShapeDtypeStruct((TOTAL * TSE_DIM // CHUNK, CHUNK), jnp.float32),
        scratch_types=[
            pltpu.VMEM((HIST, CHUNK), jnp.int32),
            pltpu.VMEM((CHUNK, TSE_DIM), jnp.float32),
            pltpu.VMEM((CHUNK, TSE_DIM), jnp.float32),
            pltpu.VMEM((TSE_DIM, Y_STRIDE), jnp.float32),
            pltpu.VMEM((TSE_DIM, Y_STRIDE), jnp.float32),
            pltpu.SemaphoreType.DMA,
            pltpu.SemaphoreType.DMA,
            pltpu.SemaphoreType.DMA,
            pltpu.SemaphoreType.DMA,
        ],
    )
    def k(idx_hbm, table_hbm, out_hbm, idx_v, g0, g1, y0, y1,
          gsem0, gsem1, wsem0, wsem1):
        wid = lax.axis_index("s") * NUM_CORES + lax.axis_index("c")
        pltpu.sync_copy(idx_hbm.at[wid], idx_v)

        iota = lax.iota(jnp.int32, 16)
        rowidx = [iota + seg * 16 for seg in range(4)]
        zeros16 = jnp.zeros((16,), jnp.int32)

        def transpose(g, y):
            def cbody(p, _):
                c0 = p * 8
                for dc in range(8):
                    c = c0 + dc
                    cvec = zeros16 + c
                    for seg in range(4):
                        v = g[c, pl.ds(seg * 16, 16)]
                        plsc.store_scatter(y, [rowidx[seg], cvec], v)
                return 0

            lax.fori_loop(0, CHUNK // 8, cbody, 0)

        def issue_writes(y, l, wsem):
            base = l * (L_STRIDE // CHUNK) + wid * (TILE_W // CHUNK)
            for i in range(8):
                pltpu.async_copy(
                    y.at[pl.ds(i * 8, 8), pl.ds(0, CHUNK)],
                    out_hbm.at[pl.ds(base + i * (I_STRIDE // CHUNK), 8)],
                    wsem,
                )

        def drain_writes(y, wsem):
            for _ in range(8):
                pltpu.make_async_copy(
                    y.at[pl.ds(0, 8), pl.ds(0, CHUNK)],
                    out_hbm.at[pl.ds(0, 8)],
                    wsem,
                ).wait()

        pltpu.async_copy(table_hbm.at[idx_v.at[0]], g0, gsem0)

        def pair_body(p, _):
            l0 = p * 2
            pltpu.make_async_copy(table_hbm.at[idx_v.at[l0]], g0, gsem0).wait()
            pltpu.async_copy(table_hbm.at[idx_v.at[l0 + 1]], g1, gsem1)

            @pl.when(p >= 1)
            def _():
                drain_writes(y0, wsem0)

            transpose(g0, y0)
            issue_writes(y0, l0, wsem0)

            pltpu.make_async_copy(
                table_hbm.at[idx_v.at[l0 + 1]], g1, gsem1
            ).wait()

            @pl.when(l0 + 2 < HIST)
            def _():
                pltpu.async_copy(table_hbm.at[idx_v.at[l0 + 2]], g0, gsem0)

            @pl.when(p >= 1)
            def _():
                drain_writes(y1, wsem1)

            transpose(g1, y1)
            issue_writes(y1, l0 + 1, wsem1)

            return 0

        lax.fori_loop(0, HIST // 2, pair_body, 0)
        drain_writes(y0, wsem0)
        drain_writes(y1, wsem1)

    return k(idx3, table)


def kernel(product_ids, semantic_table, fcn_W, fcn_b, id_table):
    wt = fcn_W.T
    b2 = fcn_b.reshape(1, TSE_DIM)
    idt = id_table.T
    table = _fused_table(semantic_table, wt, b2, idt)
    table2 = table.reshape(2 * ROWS, TSE_DIM)
    idx3 = (
        (product_ids.astype(jnp.int32) * 2).T.reshape(HIST, NW, CHUNK)
        .transpose(1, 0, 2)
    )
    out = _gather(idx3, table2)
    return (
        out.reshape(HIST, 8, NW, 8, CHUNK)
        .transpose(2, 4, 0, 1, 3)
        .reshape(BATCH, HIST, TSE_DIM)
    )
